# Initial kernel scaffold; baseline (speedup 1.0000x reference)
import functools

import jax
import jax.numpy as jnp
from jax import lax
from jax.experimental import pallas as pl
from jax.experimental.pallas import tpu as pltpu
from jax.experimental.pallas import tpu_sc as plsc

N = 10000
E = 320000
H = 128
B = 64

NC = 2
NS = 16
NPAD = 10240
EPT = E // NS
CH = 125
NCHUNK = EPT // CH
RPT = NPAD // NS

_mesh = plsc.VectorSubcoreMesh(
    core_axis_name="c", subcore_axis_name="s", num_cores=NC, num_subcores=NS
)


@functools.partial(
    pl.kernel,
    out_type=jax.ShapeDtypeStruct((2 * NPAD, H), jnp.float32),
    mesh=_mesh,
    scratch_types=[
        pltpu.VMEM((NCHUNK, CH), jnp.int32),
        pltpu.VMEM((NCHUNK, CH), jnp.int32),
        pltpu.VMEM((CH, H), jnp.float32),
        pltpu.VMEM_SHARED((NPAD, H), jnp.float32),
        pltpu.SemaphoreType.DMA,
    ],
)
def _sc_agg(x_hbm, g_hbm, s_hbm, z_hbm, out_hbm, gidx_v, sidx_v, rows_v,
            acc_sh, sem):
    c = lax.axis_index("c")
    s = lax.axis_index("s")
    pltpu.sync_copy(z_hbm.at[pl.ds(s * RPT, RPT)],
                    acc_sh.at[pl.ds(s * RPT, RPT)])
    pltpu.sync_copy(g_hbm.at[c, s], gidx_v)
    pltpu.sync_copy(s_hbm.at[c, s], sidx_v)
    plsc.subcore_barrier()

    @pl.loop(0, NCHUNK)
    def _chunk(j):
        pltpu.async_copy(x_hbm.at[gidx_v.at[j]], rows_v, sem).wait()
        pltpu.sync_copy(rows_v, acc_sh.at[sidx_v.at[j]], add=True)

    plsc.subcore_barrier()
    pltpu.sync_copy(acc_sh.at[pl.ds(s * RPT, RPT)],
                    out_hbm.at[pl.ds(c * NPAD + s * RPT, RPT)])


def _tc_body(x_r, ad_r, au_r, eps_r, dW1_r, dlns_r, dlnb_r, dW2_r, uW1_r,
             ulns_r, ulnb_r, uW2_r, l1s_r, l1b_r, l2s_r, l2b_r, de_r, cWd_r,
             cWu_r, cb_r, o_r):
    x = x_r[...]
    hp = jax.lax.Precision.HIGHEST

    def ln(h, s_, b_):
        m = jnp.mean(h, axis=-1, keepdims=True)
        v = jnp.mean((h - m) * (h - m), axis=-1, keepdims=True)
        return (h - m) * jax.lax.rsqrt(v + 1e-5) * s_ + b_

    def gin(agg, eps, W1, lns, lnb, W2):
        h = (1.0 + eps) * x + agg
        h = jnp.dot(h, W1, preferred_element_type=jnp.float32, precision=hp)
        h = jnp.maximum(ln(h, lns, lnb), 0.0)
        return jnp.dot(h, W2, preferred_element_type=jnp.float32, precision=hp)

    hd = gin(ad_r[...], eps_r[0, 0], dW1_r[...], dlns_r[...], dlnb_r[...],
             dW2_r[...])
    hd = ln(jnp.maximum(hd + de_r[0:1, :], 0.0), l1s_r[...], l1b_r[...])
    hu = gin(au_r[...], eps_r[0, 1], uW1_r[...], ulns_r[...], ulnb_r[...],
             uW2_r[...])
    hu = ln(jnp.maximum(hu + de_r[1:2, :], 0.0), l2s_r[...], l2b_r[...])
    o_r[...] = (
        jnp.dot(hd, cWd_r[...], preferred_element_type=jnp.float32,
                precision=hp)
        + jnp.dot(hu, cWu_r[...], preferred_element_type=jnp.float32,
                  precision=hp)
        + cb_r[...]
    )


def kernel(x, edge_index, down_W1, down_lns, down_lnb, down_W2, down_eps,
           up_W1, up_lns, up_lnb, up_W2, up_eps, ln1_s, ln1_b, ln2_s, ln2_b,
           dir_emb, comb_W, comb_b):
    src_dst = edge_index
    dst_src = edge_index[::-1]
    g_idx = src_dst.reshape(2, NS, NCHUNK, CH)
    s_idx = dst_src.reshape(2, NS, NCHUNK, CH)
    zeros = jnp.zeros((NPAD, H), jnp.float32)

    aggs = _sc_agg(x, g_idx, s_idx, zeros)
    ad = aggs[:NPAD]
    au = aggs[NPAD:]

    x_pad = jnp.concatenate([x, jnp.zeros((NPAD - N, H), jnp.float32)], axis=0)
    eps2 = jnp.stack([down_eps, up_eps]).reshape(1, 2)

    RB = 512
    grid = NPAD // RB
    row_spec = pl.BlockSpec((RB, H), lambda i: (i, 0))

    def full(shape):
        return pl.BlockSpec(shape, lambda i: tuple(0 for _ in shape))

    out = pl.pallas_call(
        _tc_body,
        grid=(grid,),
        in_specs=[
            row_spec, row_spec, row_spec,
            full((1, 2)),
            full((H, B)), full((1, B)), full((1, B)), full((B, H)),
            full((H, B)), full((1, B)), full((1, B)), full((B, H)),
            full((1, H)), full((1, H)), full((1, H)), full((1, H)),
            full((2, H)),
            full((H, H)), full((H, H)), full((1, H)),
        ],
        out_specs=row_spec,
        out_shape=jax.ShapeDtypeStruct((NPAD, H), jnp.float32),
    )(
        x_pad, ad, au, eps2,
        down_W1, down_lns.reshape(1, B), down_lnb.reshape(1, B), down_W2,
        up_W1, up_lns.reshape(1, B), up_lnb.reshape(1, B), up_W2,
        ln1_s.reshape(1, H), ln1_b.reshape(1, H),
        ln2_s.reshape(1, H), ln2_b.reshape(1, H),
        dir_emb,
        comb_W[:H], comb_W[H:], comb_b.reshape(1, H),
    )
    return out[:N]


# speedup vs baseline: 1.7377x; 1.7377x over previous
import functools

import jax
import jax.numpy as jnp
from jax import lax
from jax.experimental import pallas as pl
from jax.experimental.pallas import tpu as pltpu
from jax.experimental.pallas import tpu_sc as plsc

N = 10000
E = 320000
H = 128
B = 64

NC = 2
NS = 16
NPAD = 10240
EPT = E // NS
CH = 125
NCHUNK = EPT // CH
RPT = NPAD // NS

_mesh = plsc.VectorSubcoreMesh(
    core_axis_name="c", subcore_axis_name="s", num_cores=NC, num_subcores=NS
)


@functools.partial(
    pl.kernel,
    out_type=jax.ShapeDtypeStruct((2 * NPAD, H), jnp.float32),
    mesh=_mesh,
    scratch_types=[
        pltpu.VMEM((2, CH), jnp.int32),
        pltpu.VMEM((CH, H), jnp.float32),
        pltpu.VMEM_SHARED((NPAD, H), jnp.float32),
        pltpu.SemaphoreType.DMA,
    ],
)
def _sc_agg(x_hbm, gs_hbm, z_hbm, out_hbm, gs_v, rows_v, acc_sh, sem):
    c = lax.axis_index("c")
    s = lax.axis_index("s")
    pltpu.sync_copy(z_hbm.at[pl.ds(s * RPT, RPT)],
                    acc_sh.at[pl.ds(s * RPT, RPT)])
    plsc.subcore_barrier()

    @pl.loop(0, NCHUNK)
    def _chunk(j):
        pltpu.sync_copy(gs_hbm.at[c, s, j], gs_v)
        pltpu.async_copy(x_hbm.at[gs_v.at[0]], rows_v, sem).wait()
        pltpu.sync_copy(rows_v, acc_sh.at[gs_v.at[1]], add=True)

    plsc.subcore_barrier()
    pltpu.sync_copy(acc_sh.at[pl.ds(s * RPT, RPT)],
                    out_hbm.at[pl.ds(c * NPAD + s * RPT, RPT)])


def _tc_body(x_r, ad_r, au_r, eps_r, dW1_r, dlns_r, dlnb_r, dW2_r, uW1_r,
             ulns_r, ulnb_r, uW2_r, l1s_r, l1b_r, l2s_r, l2b_r, de_r, cWd_r,
             cWu_r, cb_r, o_r):
    x = x_r[...]
    hp = jax.lax.Precision.HIGHEST

    def ln(h, s_, b_):
        m = jnp.mean(h, axis=-1, keepdims=True)
        v = jnp.mean((h - m) * (h - m), axis=-1, keepdims=True)
        return (h - m) * jax.lax.rsqrt(v + 1e-5) * s_ + b_

    def gin(agg, eps, W1, lns, lnb, W2):
        h = (1.0 + eps) * x + agg
        h = jnp.dot(h, W1, preferred_element_type=jnp.float32, precision=hp)
        h = jnp.maximum(ln(h, lns, lnb), 0.0)
        return jnp.dot(h, W2, preferred_element_type=jnp.float32, precision=hp)

    hd = gin(ad_r[...], eps_r[0, 0], dW1_r[...], dlns_r[...], dlnb_r[...],
             dW2_r[...])
    hd = ln(jnp.maximum(hd + de_r[0:1, :], 0.0), l1s_r[...], l1b_r[...])
    hu = gin(au_r[...], eps_r[0, 1], uW1_r[...], ulns_r[...], ulnb_r[...],
             uW2_r[...])
    hu = ln(jnp.maximum(hu + de_r[1:2, :], 0.0), l2s_r[...], l2b_r[...])
    o_r[...] = (
        jnp.dot(hd, cWd_r[...], preferred_element_type=jnp.float32,
                precision=hp)
        + jnp.dot(hu, cWu_r[...], preferred_element_type=jnp.float32,
                  precision=hp)
        + cb_r[...]
    )


def kernel(x, edge_index, down_W1, down_lns, down_lnb, down_W2, down_eps,
           up_W1, up_lns, up_lnb, up_W2, up_eps, ln1_s, ln1_b, ln2_s, ln2_b,
           dir_emb, comb_W, comb_b):
    g_idx = edge_index.reshape(2, NS, NCHUNK, CH)
    s_idx = edge_index[::-1].reshape(2, NS, NCHUNK, CH)
    gs_idx = jnp.stack([g_idx, s_idx], axis=3)
    zeros = jnp.zeros((NPAD, H), jnp.float32)

    aggs = _sc_agg(x, gs_idx, zeros)
    ad = aggs[:NPAD]
    au = aggs[NPAD:]

    x_pad = jnp.concatenate([x, jnp.zeros((NPAD - N, H), jnp.float32)], axis=0)
    eps2 = jnp.stack([down_eps, up_eps]).reshape(1, 2)

    RB = 512
    grid = NPAD // RB
    row_spec = pl.BlockSpec((RB, H), lambda i: (i, 0))

    def full(shape):
        return pl.BlockSpec(shape, lambda i: tuple(0 for _ in shape))

    out = pl.pallas_call(
        _tc_body,
        grid=(grid,),
        in_specs=[
            row_spec, row_spec, row_spec,
            full((1, 2)),
            full((H, B)), full((1, B)), full((1, B)), full((B, H)),
            full((H, B)), full((1, B)), full((1, B)), full((B, H)),
            full((1, H)), full((1, H)), full((1, H)), full((1, H)),
            full((2, H)),
            full((H, H)), full((H, H)), full((1, H)),
        ],
        out_specs=row_spec,
        out_shape=jax.ShapeDtypeStruct((NPAD, H), jnp.float32),
    )(
        x_pad, ad, au, eps2,
        down_W1, down_lns.reshape(1, B), down_lnb.reshape(1, B), down_W2,
        up_W1, up_lns.reshape(1, B), up_lnb.reshape(1, B), up_W2,
        ln1_s.reshape(1, H), ln1_b.reshape(1, H),
        ln2_s.reshape(1, H), ln2_b.reshape(1, H),
        dir_emb,
        comb_W[:H], comb_W[H:], comb_b.reshape(1, H),
    )
    return out[:N]


# speedup vs baseline: 2.0111x; 1.1574x over previous
import functools

import jax
import jax.numpy as jnp
from jax import lax
from jax.experimental import pallas as pl
from jax.experimental.pallas import tpu as pltpu
from jax.experimental.pallas import tpu_sc as plsc

N = 10000
E = 320000
H = 128
B = 64

NC = 2
NS = 16
NPAD = 10240
EPT = E // NS
CH = 125
NCHUNK = EPT // CH
RPT = NPAD // NS

_mesh = plsc.VectorSubcoreMesh(
    core_axis_name="c", subcore_axis_name="s", num_cores=NC, num_subcores=NS
)


@functools.partial(
    pl.kernel,
    out_type=jax.ShapeDtypeStruct((2 * NPAD, H), jnp.float32),
    mesh=_mesh,
    scratch_types=[
        [pltpu.VMEM((2, CH), jnp.int32)] * 4,
        [pltpu.VMEM((CH, H), jnp.float32)] * 2,
        pltpu.VMEM_SHARED((NPAD, H), jnp.float32),
        [pltpu.SemaphoreType.DMA] * 2,
        [pltpu.SemaphoreType.DMA] * 2,
    ],
)
def _sc_agg(x_hbm, gs_hbm, z_hbm, out_hbm, gsv, rows, acc_sh, semi, semg):
    c = lax.axis_index("c")
    s = lax.axis_index("s")
    pltpu.sync_copy(z_hbm.at[pl.ds(s * RPT, RPT)],
                    acc_sh.at[pl.ds(s * RPT, RPT)])
    plsc.subcore_barrier()

    def start_idx(j, q, p):
        pltpu.async_copy(gs_hbm.at[c, s, j], gsv[q], semi[p])

    def wait_idx(p):
        pltpu.make_async_copy(gs_hbm.at[c, s, 0], gsv[0], semi[p]).wait()

    def start_gather(q, b):
        pltpu.async_copy(x_hbm.at[gsv[q].at[0]], rows[b], semg[b])

    def wait_gather(b):
        pltpu.make_async_copy(x_hbm.at[gsv[0].at[0]], rows[b], semg[b]).wait()

    def scatter(q, b):
        pltpu.sync_copy(rows[b], acc_sh.at[gsv[q].at[1]], add=True)

    start_idx(0, 0, 0)
    start_idx(1, 1, 1)
    wait_idx(0)
    start_gather(0, 0)
    start_idx(2, 2, 0)

    @pl.loop(0, (NCHUNK - 4) // 4)
    def _grp(g):
        for t in range(4):
            j = 4 * g + t
            wait_idx((t + 1) % 2)
            start_gather((t + 1) % 4, 1 - (t % 2))
            pltpu.async_copy(gs_hbm.at[c, s, j + 3], gsv[(t + 3) % 4],
                             semi[(t + 3) % 2])
            wait_gather(t % 2)
            scatter(t % 4, t % 2)

    for j in range(NCHUNK - 4, NCHUNK):
        t = j % 4
        if j + 1 < NCHUNK:
            wait_idx((j + 1) % 2)
            start_gather((j + 1) % 4, 1 - (t % 2))
        if j + 3 < NCHUNK:
            start_idx(j + 3, (j + 3) % 4, (j + 3) % 2)
        wait_gather(t % 2)
        scatter(t % 4, t % 2)

    plsc.subcore_barrier()
    pltpu.sync_copy(acc_sh.at[pl.ds(s * RPT, RPT)],
                    out_hbm.at[pl.ds(c * NPAD + s * RPT, RPT)])


def _tc_body(x_r, ad_r, au_r, eps_r, dW1_r, dlns_r, dlnb_r, dW2_r, uW1_r,
             ulns_r, ulnb_r, uW2_r, l1s_r, l1b_r, l2s_r, l2b_r, de_r, cWd_r,
             cWu_r, cb_r, o_r):
    x = x_r[...]
    hp = jax.lax.Precision.HIGHEST

    def ln(h, s_, b_):
        m = jnp.mean(h, axis=-1, keepdims=True)
        v = jnp.mean((h - m) * (h - m), axis=-1, keepdims=True)
        return (h - m) * jax.lax.rsqrt(v + 1e-5) * s_ + b_

    def gin(agg, eps, W1, lns, lnb, W2):
        h = (1.0 + eps) * x + agg
        h = jnp.dot(h, W1, preferred_element_type=jnp.float32, precision=hp)
        h = jnp.maximum(ln(h, lns, lnb), 0.0)
        return jnp.dot(h, W2, preferred_element_type=jnp.float32, precision=hp)

    hd = gin(ad_r[...], eps_r[0, 0], dW1_r[...], dlns_r[...], dlnb_r[...],
             dW2_r[...])
    hd = ln(jnp.maximum(hd + de_r[0:1, :], 0.0), l1s_r[...], l1b_r[...])
    hu = gin(au_r[...], eps_r[0, 1], uW1_r[...], ulns_r[...], ulnb_r[...],
             uW2_r[...])
    hu = ln(jnp.maximum(hu + de_r[1:2, :], 0.0), l2s_r[...], l2b_r[...])
    o_r[...] = (
        jnp.dot(hd, cWd_r[...], preferred_element_type=jnp.float32,
                precision=hp)
        + jnp.dot(hu, cWu_r[...], preferred_element_type=jnp.float32,
                  precision=hp)
        + cb_r[...]
    )


def kernel(x, edge_index, down_W1, down_lns, down_lnb, down_W2, down_eps,
           up_W1, up_lns, up_lnb, up_W2, up_eps, ln1_s, ln1_b, ln2_s, ln2_b,
           dir_emb, comb_W, comb_b):
    g_idx = edge_index.reshape(2, NS, NCHUNK, CH)
    s_idx = edge_index[::-1].reshape(2, NS, NCHUNK, CH)
    gs_idx = jnp.stack([g_idx, s_idx], axis=3)
    zeros = jnp.zeros((NPAD, H), jnp.float32)

    aggs = _sc_agg(x, gs_idx, zeros)
    ad = aggs[:NPAD]
    au = aggs[NPAD:]

    x_pad = jnp.concatenate([x, jnp.zeros((NPAD - N, H), jnp.float32)], axis=0)
    eps2 = jnp.stack([down_eps, up_eps]).reshape(1, 2)

    RB = 512
    grid = NPAD // RB
    row_spec = pl.BlockSpec((RB, H), lambda i: (i, 0))

    def full(shape):
        return pl.BlockSpec(shape, lambda i: tuple(0 for _ in shape))

    out = pl.pallas_call(
        _tc_body,
        grid=(grid,),
        in_specs=[
            row_spec, row_spec, row_spec,
            full((1, 2)),
            full((H, B)), full((1, B)), full((1, B)), full((B, H)),
            full((H, B)), full((1, B)), full((1, B)), full((B, H)),
            full((1, H)), full((1, H)), full((1, H)), full((1, H)),
            full((2, H)),
            full((H, H)), full((H, H)), full((1, H)),
        ],
        out_specs=row_spec,
        out_shape=jax.ShapeDtypeStruct((NPAD, H), jnp.float32),
    )(
        x_pad, ad, au, eps2,
        down_W1, down_lns.reshape(1, B), down_lnb.reshape(1, B), down_W2,
        up_W1, up_lns.reshape(1, B), up_lnb.reshape(1, B), up_W2,
        ln1_s.reshape(1, H), ln1_b.reshape(1, H),
        ln2_s.reshape(1, H), ln2_b.reshape(1, H),
        dir_emb,
        comb_W[:H], comb_W[H:], comb_b.reshape(1, H),
    )
    return out[:N]


# speedup vs baseline: 2.0279x; 1.0083x over previous
import functools

import jax
import jax.numpy as jnp
from jax import lax
from jax.experimental import pallas as pl
from jax.experimental.pallas import tpu as pltpu
from jax.experimental.pallas import tpu_sc as plsc

N = 10000
E = 320000
H = 128
B = 64

NC = 2
NS = 16
NPAD = 10240
EPT = E // NS
CH = 125
NCHUNK = EPT // CH
RPT = NPAD // NS

_mesh = plsc.VectorSubcoreMesh(
    core_axis_name="c", subcore_axis_name="s", num_cores=NC, num_subcores=NS
)


@functools.partial(
    pl.kernel,
    out_type=jax.ShapeDtypeStruct((2 * NPAD, H), jnp.float32),
    mesh=_mesh,
    scratch_types=[
        [pltpu.VMEM((2, CH), jnp.int32)] * 4,
        [pltpu.VMEM((CH, H), jnp.float32)] * 2,
        pltpu.VMEM_SHARED((NPAD, H), jnp.float32),
        [pltpu.SemaphoreType.DMA] * 2,
        [pltpu.SemaphoreType.DMA] * 2,
    ],
)
def _sc_agg(x_hbm, gs_hbm, z_hbm, out_hbm, gsv, rows, acc_sh, semi, semg):
    c = lax.axis_index("c")
    s = lax.axis_index("s")
    pltpu.sync_copy(z_hbm.at[pl.ds(s * RPT, RPT)],
                    acc_sh.at[pl.ds(s * RPT, RPT)])
    plsc.subcore_barrier()

    def start_idx(j, q, p):
        pltpu.async_copy(gs_hbm.at[c, s, j], gsv[q], semi[p])

    def wait_idx(p):
        pltpu.make_async_copy(gs_hbm.at[c, s, 0], gsv[0], semi[p]).wait()

    def start_gather(q, b):
        pltpu.async_copy(x_hbm.at[gsv[q].at[0]], rows[b], semg[b])

    def wait_gather(b):
        pltpu.make_async_copy(x_hbm.at[gsv[0].at[0]], rows[b], semg[b]).wait()

    def scatter(q, b):
        pltpu.sync_copy(rows[b], acc_sh.at[gsv[q].at[1]], add=True)

    start_idx(0, 0, 0)
    start_idx(1, 1, 1)
    wait_idx(0)
    start_gather(0, 0)
    start_idx(2, 2, 0)

    @pl.loop(0, (NCHUNK - 4) // 4)
    def _grp(g):
        for t in range(4):
            j = 4 * g + t
            wait_idx((t + 1) % 2)
            start_gather((t + 1) % 4, 1 - (t % 2))
            pltpu.async_copy(gs_hbm.at[c, s, j + 3], gsv[(t + 3) % 4],
                             semi[(t + 3) % 2])
            wait_gather(t % 2)
            scatter(t % 4, t % 2)

    for j in range(NCHUNK - 4, NCHUNK):
        t = j % 4
        if j + 1 < NCHUNK:
            wait_idx((j + 1) % 2)
            start_gather((j + 1) % 4, 1 - (t % 2))
        if j + 3 < NCHUNK:
            start_idx(j + 3, (j + 3) % 4, (j + 3) % 2)
        wait_gather(t % 2)
        scatter(t % 4, t % 2)

    plsc.subcore_barrier()
    pltpu.sync_copy(acc_sh.at[pl.ds(s * RPT, RPT)],
                    out_hbm.at[pl.ds(c * NPAD + s * RPT, RPT)])


def _tc_body(x_r, ad_r, au_r, eps_r, dW1_r, dlns_r, dlnb_r, dW2_r, uW1_r,
             ulns_r, ulnb_r, uW2_r, l1s_r, l1b_r, l2s_r, l2b_r, de_r, cWd_r,
             cWu_r, cb_r, o_r):
    x = x_r[...]
    hp = jax.lax.Precision.HIGHEST

    def ln(h, s_, b_):
        m = jnp.mean(h, axis=-1, keepdims=True)
        v = jnp.mean((h - m) * (h - m), axis=-1, keepdims=True)
        return (h - m) * jax.lax.rsqrt(v + 1e-5) * s_ + b_

    def gin(agg, eps, W1, lns, lnb, W2):
        h = (1.0 + eps) * x + agg
        h = jnp.dot(h, W1, preferred_element_type=jnp.float32, precision=hp)
        h = jnp.maximum(ln(h, lns, lnb), 0.0)
        return jnp.dot(h, W2, preferred_element_type=jnp.float32, precision=hp)

    hd = gin(ad_r[...], eps_r[0, 0], dW1_r[...], dlns_r[...], dlnb_r[...],
             dW2_r[...])
    hd = ln(jnp.maximum(hd + de_r[0:1, :], 0.0), l1s_r[...], l1b_r[...])
    hu = gin(au_r[...], eps_r[0, 1], uW1_r[...], ulns_r[...], ulnb_r[...],
             uW2_r[...])
    hu = ln(jnp.maximum(hu + de_r[1:2, :], 0.0), l2s_r[...], l2b_r[...])
    o_r[...] = (
        jnp.dot(hd, cWd_r[...], preferred_element_type=jnp.float32,
                precision=hp)
        + jnp.dot(hu, cWu_r[...], preferred_element_type=jnp.float32,
                  precision=hp)
        + cb_r[...]
    )


def kernel(x, edge_index, down_W1, down_lns, down_lnb, down_W2, down_eps,
           up_W1, up_lns, up_lnb, up_W2, up_eps, ln1_s, ln1_b, ln2_s, ln2_b,
           dir_emb, comb_W, comb_b):
    g_idx = edge_index.reshape(2, NS, NCHUNK, CH)
    s_idx = edge_index[::-1].reshape(2, NS, NCHUNK, CH)
    gs_idx = jnp.stack([g_idx, s_idx], axis=3)
    zeros = jnp.zeros((NPAD, H), jnp.float32)

    aggs = _sc_agg(x, gs_idx, zeros)

    eps2 = jnp.stack([down_eps, up_eps]).reshape(1, 2)

    RB = 512
    grid = pl.cdiv(N, RB)
    row_spec = pl.BlockSpec((RB, H), lambda i: (i, 0))
    ad_spec = pl.BlockSpec((RB, H), lambda i: (i, 0))
    au_spec = pl.BlockSpec((RB, H), lambda i: (i + NPAD // RB, 0))

    def full(shape):
        return pl.BlockSpec(shape, lambda i: tuple(0 for _ in shape))

    out = pl.pallas_call(
        _tc_body,
        grid=(grid,),
        in_specs=[
            row_spec, ad_spec, au_spec,
            full((1, 2)),
            full((H, B)), full((1, B)), full((1, B)), full((B, H)),
            full((H, B)), full((1, B)), full((1, B)), full((B, H)),
            full((1, H)), full((1, H)), full((1, H)), full((1, H)),
            full((2, H)),
            full((H, H)), full((H, H)), full((1, H)),
        ],
        out_specs=row_spec,
        out_shape=jax.ShapeDtypeStruct((N, H), jnp.float32),
    )(
        x, aggs, aggs, eps2,
        down_W1, down_lns.reshape(1, B), down_lnb.reshape(1, B), down_W2,
        up_W1, up_lns.reshape(1, B), up_lnb.reshape(1, B), up_W2,
        ln1_s.reshape(1, H), ln1_b.reshape(1, H),
        ln2_s.reshape(1, H), ln2_b.reshape(1, H),
        dir_emb,
        comb_W[:H], comb_W[H:], comb_b.reshape(1, H),
    )
    return out


# speedup vs baseline: 2.0833x; 1.0273x over previous
import functools

import jax
import jax.numpy as jnp
from jax import lax
from jax.experimental import pallas as pl
from jax.experimental.pallas import tpu as pltpu
from jax.experimental.pallas import tpu_sc as plsc

N = 10000
E = 320000
H = 128
B = 64

NC = 2
NS = 16
NPAD = 10240
EPT = E // NS
CH = 125
NCHUNK = EPT // CH
RPT = NPAD // NS

_mesh = plsc.VectorSubcoreMesh(
    core_axis_name="c", subcore_axis_name="s", num_cores=NC, num_subcores=NS
)


@functools.partial(
    pl.kernel,
    out_type=jax.ShapeDtypeStruct((2 * NPAD, H), jnp.float32),
    mesh=_mesh,
    scratch_types=[
        [pltpu.VMEM((2, CH), jnp.int32)] * 4,
        [pltpu.VMEM((CH, H), jnp.float32)] * 2,
        pltpu.VMEM_SHARED((NPAD, H), jnp.float32),
        [pltpu.SemaphoreType.DMA] * 2,
        [pltpu.SemaphoreType.DMA] * 2,
    ],
)
def _sc_agg(x_hbm, gs_hbm, z_hbm, out_hbm, gsv, rows, acc_sh, semi, semg):
    c = lax.axis_index("c")
    s = lax.axis_index("s")
    pltpu.sync_copy(z_hbm.at[pl.ds(s * RPT, RPT)],
                    acc_sh.at[pl.ds(s * RPT, RPT)])
    plsc.subcore_barrier()

    def start_idx(j, q, p):
        pltpu.async_copy(gs_hbm.at[c, s, j], gsv[q], semi[p])

    def wait_idx(p):
        pltpu.make_async_copy(gs_hbm.at[c, s, 0], gsv[0], semi[p]).wait()

    def start_gather(q, b):
        pltpu.async_copy(x_hbm.at[gsv[q].at[0]], rows[b], semg[b])

    def wait_gather(b):
        pltpu.make_async_copy(x_hbm.at[gsv[0].at[0]], rows[b], semg[b]).wait()

    def scatter(q, b):
        pltpu.sync_copy(rows[b], acc_sh.at[gsv[q].at[1]], add=True)

    start_idx(0, 0, 0)
    start_idx(1, 1, 1)
    wait_idx(0)
    start_gather(0, 0)
    start_idx(2, 2, 0)

    @pl.loop(0, (NCHUNK - 4) // 4)
    def _grp(g):
        for t in range(4):
            j = 4 * g + t
            wait_idx((t + 1) % 2)
            start_gather((t + 1) % 4, 1 - (t % 2))
            pltpu.async_copy(gs_hbm.at[c, s, j + 3], gsv[(t + 3) % 4],
                             semi[(t + 3) % 2])
            wait_gather(t % 2)
            scatter(t % 4, t % 2)

    for j in range(NCHUNK - 4, NCHUNK):
        t = j % 4
        if j + 1 < NCHUNK:
            wait_idx((j + 1) % 2)
            start_gather((j + 1) % 4, 1 - (t % 2))
        if j + 3 < NCHUNK:
            start_idx(j + 3, (j + 3) % 4, (j + 3) % 2)
        wait_gather(t % 2)
        scatter(t % 4, t % 2)

    plsc.subcore_barrier()
    pltpu.sync_copy(acc_sh.at[pl.ds(s * RPT, RPT)],
                    out_hbm.at[pl.ds(c * NPAD + s * RPT, RPT)])


def _tc_body(x_r, ad_r, au_r, eps_r, dW1_r, dlns_r, dlnb_r, dW2_r, uW1_r,
             ulns_r, ulnb_r, uW2_r, l1s_r, l1b_r, l2s_r, l2b_r, de_r, cWd_r,
             cWu_r, cb_r, o_r):
    x = x_r[...]
    hp = jax.lax.Precision.DEFAULT

    def ln(h, s_, b_):
        m = jnp.mean(h, axis=-1, keepdims=True)
        v = jnp.mean((h - m) * (h - m), axis=-1, keepdims=True)
        return (h - m) * jax.lax.rsqrt(v + 1e-5) * s_ + b_

    def gin(agg, eps, W1, lns, lnb, W2):
        h = (1.0 + eps) * x + agg
        h = jnp.dot(h, W1, preferred_element_type=jnp.float32, precision=hp)
        h = jnp.maximum(ln(h, lns, lnb), 0.0)
        return jnp.dot(h, W2, preferred_element_type=jnp.float32, precision=hp)

    hd = gin(ad_r[...], eps_r[0, 0], dW1_r[...], dlns_r[...], dlnb_r[...],
             dW2_r[...])
    hd = ln(jnp.maximum(hd + de_r[0:1, :], 0.0), l1s_r[...], l1b_r[...])
    hu = gin(au_r[...], eps_r[0, 1], uW1_r[...], ulns_r[...], ulnb_r[...],
             uW2_r[...])
    hu = ln(jnp.maximum(hu + de_r[1:2, :], 0.0), l2s_r[...], l2b_r[...])
    o_r[...] = (
        jnp.dot(hd, cWd_r[...], preferred_element_type=jnp.float32,
                precision=hp)
        + jnp.dot(hu, cWu_r[...], preferred_element_type=jnp.float32,
                  precision=hp)
        + cb_r[...]
    )


def kernel(x, edge_index, down_W1, down_lns, down_lnb, down_W2, down_eps,
           up_W1, up_lns, up_lnb, up_W2, up_eps, ln1_s, ln1_b, ln2_s, ln2_b,
           dir_emb, comb_W, comb_b):
    g_idx = edge_index.reshape(2, NS, NCHUNK, CH)
    s_idx = edge_index[::-1].reshape(2, NS, NCHUNK, CH)
    gs_idx = jnp.stack([g_idx, s_idx], axis=3)
    zeros = jnp.zeros((NPAD, H), jnp.float32)

    aggs = _sc_agg(x, gs_idx, zeros)

    eps2 = jnp.stack([down_eps, up_eps]).reshape(1, 2)

    RB = 512
    grid = pl.cdiv(N, RB)
    row_spec = pl.BlockSpec((RB, H), lambda i: (i, 0))
    ad_spec = pl.BlockSpec((RB, H), lambda i: (i, 0))
    au_spec = pl.BlockSpec((RB, H), lambda i: (i + NPAD // RB, 0))

    def full(shape):
        return pl.BlockSpec(shape, lambda i: tuple(0 for _ in shape))

    out = pl.pallas_call(
        _tc_body,
        grid=(grid,),
        in_specs=[
            row_spec, ad_spec, au_spec,
            full((1, 2)),
            full((H, B)), full((1, B)), full((1, B)), full((B, H)),
            full((H, B)), full((1, B)), full((1, B)), full((B, H)),
            full((1, H)), full((1, H)), full((1, H)), full((1, H)),
            full((2, H)),
            full((H, H)), full((H, H)), full((1, H)),
        ],
        out_specs=row_spec,
        out_shape=jax.ShapeDtypeStruct((N, H), jnp.float32),
    )(
        x, aggs, aggs, eps2,
        down_W1, down_lns.reshape(1, B), down_lnb.reshape(1, B), down_W2,
        up_W1, up_lns.reshape(1, B), up_lnb.reshape(1, B), up_W2,
        ln1_s.reshape(1, H), ln1_b.reshape(1, H),
        ln2_s.reshape(1, H), ln2_b.reshape(1, H),
        dir_emb,
        comb_W[:H], comb_W[H:], comb_b.reshape(1, H),
    )
    return out
